# Initial kernel scaffold; baseline (speedup 1.0000x reference)
import functools

import jax
import jax.numpy as jnp
from jax import lax
from jax.experimental import pallas as pl
from jax.experimental.pallas import tpu as pltpu
from jax.experimental.pallas import tpu_sc as plsc

N, E, D, PC, NH = 10000, 160000, 256, 256, 512
HALF = 128
NC, NS = 2, 16
K = 128
EPT = -(-E // (NS * K)) * K
EPAD = EPT * NS
NCHUNK = EPT // K
ROWS_PT = N // NS
APAD = 8


def _sc_segment_sum(x2, xh, gidx, didx):
    mesh = plsc.VectorSubcoreMesh(core_axis_name="c", subcore_axis_name="s")

    @functools.partial(
        pl.kernel,
        mesh=mesh,
        out_type=jax.ShapeDtypeStruct((NC, N, HALF), jnp.float32),
        scratch_types=[
            pltpu.VMEM((K,), jnp.int32),
            pltpu.VMEM((1, K), jnp.int32),
            pltpu.VMEM((K, HALF), jnp.float32),
            pltpu.VMEM_SHARED((N + APAD, HALF), jnp.float32),
            pltpu.SemaphoreType.DMA,
        ],
    )
    def k(x2_hbm, xh_hbm, gidx_hbm, didx_hbm, out_hbm,
          gbuf, dbuf, rows, agg, sem):
        c = lax.axis_index("c")
        s = lax.axis_index("s")
        r0 = s * ROWS_PT
        pltpu.sync_copy(xh_hbm.at[c, pl.ds(r0, ROWS_PT)],
                        agg.at[pl.ds(r0, ROWS_PT)])
        plsc.subcore_barrier()
        base = s * EPT

        def chunk(j, carry):
            off = base + j * K
            pltpu.sync_copy(gidx_hbm.at[c, pl.ds(off, K)], gbuf)
            pltpu.sync_copy(didx_hbm.at[pl.ds(off, K)], dbuf.at[0])
            pltpu.async_copy(x2_hbm.at[gbuf], rows, sem).wait()
            pltpu.sync_copy(rows, agg.at[dbuf.at[0]], add=True)
            return carry

        lax.fori_loop(0, NCHUNK, chunk, 0)
        plsc.subcore_barrier()
        pltpu.sync_copy(agg.at[pl.ds(r0, ROWS_PT)],
                        out_hbm.at[c, pl.ds(r0, ROWS_PT)])

    return k(x2, xh, gidx, didx)


BLK = 1000


def _tc_body(a_ref, wg_ref, bg_ref, w0_ref, b0_ref, w1_ref, b1_ref,
             emb_ref, dlt_ref):
    dn = (((1,), (0,)), ((), ()))
    h0 = a_ref[0]
    h1 = a_ref[1]
    acc = lax.dot_general(h0, wg_ref[:HALF, :], dn,
                          preferred_element_type=jnp.float32,
                          precision=lax.Precision.HIGHEST)
    acc = acc + lax.dot_general(h1, wg_ref[HALF:, :], dn,
                                preferred_element_type=jnp.float32,
                                precision=lax.Precision.HIGHEST)
    e = jnp.maximum(acc + bg_ref[...], 0.0)
    emb_ref[...] = e
    t = lax.dot_general(e, w0_ref[...], dn,
                        preferred_element_type=jnp.float32,
                        precision=lax.Precision.HIGHEST) + b0_ref[...]
    t = jnp.where(t >= 0.0, t, 0.01 * t)
    dlt_ref[...] = lax.dot_general(t, w1_ref[...], dn,
                                   preferred_element_type=jnp.float32,
                                   precision=lax.Precision.HIGHEST) + b1_ref[...]


def _tc_dense(agg2, W_gcn, b_gcn, W0, b0, W1, b1):
    return pl.pallas_call(
        _tc_body,
        grid=(N // BLK,),
        in_specs=[
            pl.BlockSpec((NC, BLK, HALF), lambda i: (0, i, 0)),
            pl.BlockSpec((D, PC), lambda i: (0, 0)),
            pl.BlockSpec((1, PC), lambda i: (0, 0)),
            pl.BlockSpec((PC, NH), lambda i: (0, 0)),
            pl.BlockSpec((1, NH), lambda i: (0, 0)),
            pl.BlockSpec((NH, PC), lambda i: (0, 0)),
            pl.BlockSpec((1, PC), lambda i: (0, 0)),
        ],
        out_specs=(
            pl.BlockSpec((BLK, PC), lambda i: (i, 0)),
            pl.BlockSpec((BLK, PC), lambda i: (i, 0)),
        ),
        out_shape=(
            jax.ShapeDtypeStruct((N, PC), jnp.float32),
            jax.ShapeDtypeStruct((N, PC), jnp.float32),
        ),
    )(agg2, W_gcn, b_gcn.reshape(1, PC), W0, b0.reshape(1, NH),
      W1, b1.reshape(1, PC))


def kernel(x, edge_index, W_gcn, b_gcn, W0, b0, W1, b1):
    ei = edge_index.astype(jnp.int32)
    src, dst = ei[0], ei[1]
    pad = EPAD - E
    gidx = jnp.stack([2 * src, 2 * src + 1])
    gidx = jnp.pad(gidx, ((0, 0), (0, pad)))
    didx = jnp.pad(dst, (0, pad), constant_values=N)
    x2 = x.reshape(2 * N, HALF)
    xh = x.reshape(N, 2, HALF).transpose(1, 0, 2)
    agg2 = _sc_segment_sum(x2, xh, gidx, didx)
    x_embedding, x_delta_hat = _tc_dense(agg2, W_gcn, b_gcn, W0, b0, W1, b1)
    return (x_embedding, x_delta_hat)


# speedup vs baseline: 2.8647x; 2.8647x over previous
import functools

import jax
import jax.numpy as jnp
from jax import lax
from jax.experimental import pallas as pl
from jax.experimental.pallas import tpu as pltpu
from jax.experimental.pallas import tpu_sc as plsc

N, E, D, PC, NH = 10000, 160000, 256, 256, 512
HALF = 128
NC, NS = 2, 16
K = 128
EPT = -(-E // (NS * K)) * K
EPAD = EPT * NS
NCHUNK = EPT // K
ROWS_PT = (N // NS) // 8 * 8
TAIL = N - ROWS_PT * NS
APAD = 8


def _sc_segment_sum(x2, xh, gidx, didx):
    mesh = plsc.VectorSubcoreMesh(core_axis_name="c", subcore_axis_name="s")

    @functools.partial(
        pl.kernel,
        mesh=mesh,
        out_type=jax.ShapeDtypeStruct((NC, N, HALF), jnp.float32),
        scratch_types=[
            pltpu.VMEM((K,), jnp.int32),
            pltpu.VMEM((1, K), jnp.int32),
            pltpu.VMEM((K, HALF), jnp.float32),
            pltpu.VMEM_SHARED((N + APAD, HALF), jnp.float32),
            pltpu.SemaphoreType.DMA,
        ],
    )
    def k(x2_hbm, xh_hbm, gidx_hbm, didx_hbm, out_hbm,
          gbuf, dbuf, rows, agg, sem):
        c = lax.axis_index("c")
        s = lax.axis_index("s")
        r0 = s * ROWS_PT
        pltpu.sync_copy(xh_hbm.at[c, pl.ds(r0, ROWS_PT)],
                        agg.at[pl.ds(r0, ROWS_PT)])

        @pl.when(s == 0)
        def _():
            pltpu.sync_copy(xh_hbm.at[c, pl.ds(ROWS_PT * NS, TAIL)],
                            agg.at[pl.ds(ROWS_PT * NS, TAIL)])

        plsc.subcore_barrier()
        base = s * EPT

        def chunk(j, carry):
            off = base + j * K
            pltpu.sync_copy(gidx_hbm.at[c, pl.ds(off, K)], gbuf)
            pltpu.sync_copy(didx_hbm.at[pl.ds(off, K)], dbuf.at[0])
            pltpu.async_copy(x2_hbm.at[gbuf], rows, sem).wait()
            pltpu.sync_copy(rows, agg.at[dbuf.at[0]], add=True)
            return carry

        lax.fori_loop(0, NCHUNK, chunk, 0)
        plsc.subcore_barrier()
        pltpu.sync_copy(agg.at[pl.ds(r0, ROWS_PT)],
                        out_hbm.at[c, pl.ds(r0, ROWS_PT)])

        @pl.when(s == 0)
        def _():
            pltpu.sync_copy(agg.at[pl.ds(ROWS_PT * NS, TAIL)],
                            out_hbm.at[c, pl.ds(ROWS_PT * NS, TAIL)])

    return k(x2, xh, gidx, didx)


BLK = 1000


def _tc_body(a_ref, wg_ref, bg_ref, w0_ref, b0_ref, w1_ref, b1_ref,
             emb_ref, dlt_ref):
    dn = (((1,), (0,)), ((), ()))
    h0 = a_ref[0]
    h1 = a_ref[1]
    acc = lax.dot_general(h0, wg_ref[:HALF, :], dn,
                          preferred_element_type=jnp.float32,
                          precision=lax.Precision.HIGHEST)
    acc = acc + lax.dot_general(h1, wg_ref[HALF:, :], dn,
                                preferred_element_type=jnp.float32,
                                precision=lax.Precision.HIGHEST)
    e = jnp.maximum(acc + bg_ref[...], 0.0)
    emb_ref[...] = e
    t = lax.dot_general(e, w0_ref[...], dn,
                        preferred_element_type=jnp.float32,
                        precision=lax.Precision.HIGHEST) + b0_ref[...]
    t = jnp.where(t >= 0.0, t, 0.01 * t)
    dlt_ref[...] = lax.dot_general(t, w1_ref[...], dn,
                                   preferred_element_type=jnp.float32,
                                   precision=lax.Precision.HIGHEST) + b1_ref[...]


def _tc_dense(agg2, W_gcn, b_gcn, W0, b0, W1, b1):
    return pl.pallas_call(
        _tc_body,
        grid=(N // BLK,),
        in_specs=[
            pl.BlockSpec((NC, BLK, HALF), lambda i: (0, i, 0)),
            pl.BlockSpec((D, PC), lambda i: (0, 0)),
            pl.BlockSpec((1, PC), lambda i: (0, 0)),
            pl.BlockSpec((PC, NH), lambda i: (0, 0)),
            pl.BlockSpec((1, NH), lambda i: (0, 0)),
            pl.BlockSpec((NH, PC), lambda i: (0, 0)),
            pl.BlockSpec((1, PC), lambda i: (0, 0)),
        ],
        out_specs=(
            pl.BlockSpec((BLK, PC), lambda i: (i, 0)),
            pl.BlockSpec((BLK, PC), lambda i: (i, 0)),
        ),
        out_shape=(
            jax.ShapeDtypeStruct((N, PC), jnp.float32),
            jax.ShapeDtypeStruct((N, PC), jnp.float32),
        ),
    )(agg2, W_gcn, b_gcn.reshape(1, PC), W0, b0.reshape(1, NH),
      W1, b1.reshape(1, PC))


def kernel(x, edge_index, W_gcn, b_gcn, W0, b0, W1, b1):
    ei = edge_index.astype(jnp.int32)
    src, dst = ei[0], ei[1]
    pad = EPAD - E
    gidx = jnp.stack([2 * src, 2 * src + 1])
    gidx = jnp.pad(gidx, ((0, 0), (0, pad)))
    didx = jnp.pad(dst, (0, pad), constant_values=N)
    x2 = x.reshape(2 * N, HALF)
    xh = x.reshape(N, 2, HALF).transpose(1, 0, 2)
    agg2 = _sc_segment_sum(x2, xh, gidx, didx)
    x_embedding, x_delta_hat = _tc_dense(agg2, W_gcn, b_gcn, W0, b0, W1, b1)
    return (x_embedding, x_delta_hat)
